# baseline (speedup 1.0000x reference)
import functools

import jax
import jax.numpy as jnp
from jax import lax
from jax.experimental import pallas as pl
from jax.experimental.pallas import tpu as pltpu
from jax.experimental.pallas import tpu_sc as plsc

N = 10000
E = 320000
D = 128

NC = 2
NS = 16
NW = NC * NS
EPW = E // NW
C = 80
NCHUNK = EPW // C
NP = 10240
DPT = NP // NS
RPT = NP // NS
RZ = 16


def _build_deg_kernel(mesh):
    @functools.partial(
        pl.kernel,
        out_type=[
            jax.ShapeDtypeStruct((NC, NP), jnp.float32),
            jax.ShapeDtypeStruct((NC, NP), jnp.float32),
        ],
        mesh=mesh,
        scratch_types=[
            pltpu.VMEM((NCHUNK, C), jnp.int32),
            pltpu.VMEM((NCHUNK, C), jnp.int32),
            pltpu.VMEM((C,), jnp.float32),
            pltpu.VMEM((DPT,), jnp.float32),
            pltpu.VMEM_SHARED((NP,), jnp.float32),
            pltpu.VMEM_SHARED((NP,), jnp.float32),
            [pltpu.SemaphoreType.DMA for _ in range(3)],
        ],
    )
    def deg_kernel(src_hbm, dst_hbm, dego_hbm, degi_hbm,
                   src_v, dst_v, ones_v, z_v, dego_sp, degi_sp, sd):
        c = lax.axis_index("c")
        s = lax.axis_index("s")
        wid = c * NS + s

        pltpu.async_copy(src_hbm.at[wid], src_v, sd[2])
        pltpu.async_copy(dst_hbm.at[wid], dst_v, sd[2])

        for j in range(C // 16):
            ones_v[pl.ds(j * 16, 16)] = jnp.ones((16,), jnp.float32)

        def zero_body(i, _):
            z_v[pl.ds(i * 16, 16)] = jnp.zeros((16,), jnp.float32)
            return 0

        lax.fori_loop(0, DPT // 16, zero_body, 0)
        pltpu.sync_copy(z_v, dego_sp.at[pl.ds(s * DPT, DPT)])
        pltpu.sync_copy(z_v, degi_sp.at[pl.ds(s * DPT, DPT)])
        pltpu.make_async_copy(src_hbm.at[0], src_v, sd[2]).wait()
        pltpu.make_async_copy(src_hbm.at[0], dst_v, sd[2]).wait()
        plsc.subcore_barrier()

        def scat2_start(i, p):
            pltpu.async_copy(ones_v, dego_sp.at[src_v.at[i]], sd[p],
                             add=True)
            pltpu.async_copy(ones_v, degi_sp.at[dst_v.at[i]], sd[p],
                             add=True)

        def scat2_wait(p):
            pltpu.make_async_copy(ones_v, dego_sp.at[pl.ds(0, C)],
                                  sd[p]).wait()
            pltpu.make_async_copy(ones_v, degi_sp.at[pl.ds(0, C)],
                                  sd[p]).wait()

        scat2_start(0, 0)
        scat2_start(1, 1)

        def body(k, _):
            g = 2 * k
            scat2_wait(0)
            scat2_start(g + 2, 0)
            scat2_wait(1)
            scat2_start(g + 3, 1)
            return 0

        lax.fori_loop(0, (NCHUNK - 3) // 2, body, 0)
        scat2_wait(0)
        scat2_start(NCHUNK - 1, 0)
        scat2_wait(1)
        scat2_wait(0)
        plsc.subcore_barrier()

        pltpu.sync_copy(dego_sp.at[pl.ds(s * DPT, DPT)],
                        dego_hbm.at[c, pl.ds(s * DPT, DPT)])
        pltpu.sync_copy(degi_sp.at[pl.ds(s * DPT, DPT)],
                        degi_hbm.at[c, pl.ds(s * DPT, DPT)])

    return deg_kernel


def _build_agg_kernel(mesh):
    @functools.partial(
        pl.kernel,
        out_type=jax.ShapeDtypeStruct((NC, NP, D), jnp.float32),
        mesh=mesh,
        scratch_types=[
            [pltpu.VMEM((2, C), jnp.int32) for _ in range(4)],
            [pltpu.VMEM((C, D), jnp.float32) for _ in range(2)],
            pltpu.VMEM((RZ, D), jnp.float32),
            pltpu.VMEM_SHARED((NP, D), jnp.float32),
            [pltpu.SemaphoreType.DMA for _ in range(4)],
            [pltpu.SemaphoreType.DMA for _ in range(2)],
            [pltpu.SemaphoreType.DMA for _ in range(2)],
        ],
    )
    def agg_kernel(hn_hbm, eidx_hbm, out_hbm,
                   idx, rows, z_v, agg_sp, si, sg, ss):
        c = lax.axis_index("c")
        s = lax.axis_index("s")
        wid = c * NS + s

        def idx_start(i, r):
            pltpu.async_copy(eidx_hbm.at[wid, i], idx[r], si[r])

        def idx_wait(r):
            pltpu.make_async_copy(eidx_hbm.at[0, 0], idx[r], si[r]).wait()

        def gather_start(r_idx, p):
            pltpu.async_copy(hn_hbm.at[idx[r_idx].at[0]], rows[p], sg[p])

        def gather_wait(p):
            pltpu.make_async_copy(hn_hbm.at[pl.ds(0, C)], rows[p], sg[p]).wait()

        def scat_start(r_idx, p):
            pltpu.async_copy(rows[p], agg_sp.at[idx[r_idx].at[1]], ss[p],
                             add=True)

        def scat_wait(p):
            pltpu.make_async_copy(rows[p], agg_sp.at[pl.ds(0, C)],
                                  ss[p]).wait()

        def step(g, r, extra=0):
            p = r % 2
            scat_wait(1 - p)
            if extra:
                idx_start(g + 3, (r + 3) % 4)
            idx_wait((r + 1) % 4)
            gather_start((r + 1) % 4, 1 - p)
            gather_wait(p)
            scat_start(r, p)

        def zero_body(i, _):
            for j in range(D // 16):
                z_v[i, pl.ds(j * 16, 16)] = jnp.zeros((16,), jnp.float32)
            return 0

        lax.fori_loop(0, RZ, zero_body, 0)
        for k in range(RPT // RZ):
            pltpu.sync_copy(z_v, agg_sp.at[pl.ds(s * RPT + k * RZ, RZ)])
        plsc.subcore_barrier()

        idx_start(0, 0)
        idx_start(1, 1)
        idx_start(2, 2)
        idx_wait(0)
        gather_start(0, 0)
        idx_start(3, 3)
        idx_wait(1)
        gather_start(1, 1)
        gather_wait(0)
        scat_start(0, 0)
        step(1, 1, extra=1)
        step(2, 2, extra=1)
        step(3, 3, extra=1)

        def body(k, _):
            q = 4 * k
            step(q + 0, 0, extra=1)
            step(q + 1, 1, extra=1)
            step(q + 2, 2, extra=1)
            step(q + 3, 3, extra=1)
            return 0

        lax.fori_loop(1, (NCHUNK - 5) // 4, body, 0)
        step(NCHUNK - 5, 0, extra=1)
        step(NCHUNK - 4, 1, extra=1)
        step(NCHUNK - 3, 2)
        step(NCHUNK - 2, 3)
        scat_wait(1)
        gather_wait(0)
        scat_start(0, 0)
        scat_wait(0)

        plsc.subcore_barrier()
        pltpu.sync_copy(agg_sp.at[pl.ds(s * RPT, RPT)],
                        out_hbm.at[c, pl.ds(s * RPT, RPT)])

    return agg_kernel


@functools.cache
def _sc_kernels():
    mesh = plsc.VectorSubcoreMesh(
        core_axis_name="c", subcore_axis_name="s", num_cores=NC, num_subcores=NS
    )
    return _build_deg_kernel(mesh), _build_agg_kernel(mesh)


R = 2048
_GRID = NP // R


def _mm_body(x_ref, w1_ref, h_ref):
    h_ref[...] = jnp.dot(x_ref[...], w1_ref[...],
                         preferred_element_type=jnp.float32)


def _scale_body(h_ref, dego_ref, degi_ref, hn_ref, nsrc_ref, ndst_ref):
    dego = dego_ref[0] + dego_ref[1]
    degi = degi_ref[0] + degi_ref[1]
    nsrc = lax.rsqrt(jnp.maximum(dego, 1.0))
    ndst = lax.rsqrt(jnp.maximum(degi, 1.0))
    nsrc_ref[...] = nsrc
    ndst_ref[...] = ndst
    hn_ref[...] = h_ref[...] * nsrc


def _mid_body(agg_ref, ndst_ref, b1_ref, w2_ref, nsrc_ref, hn_ref):
    a = agg_ref[0] + agg_ref[1]
    t = a * ndst_ref[...] + b1_ref[...]
    t = jnp.where(t >= 0.0, t, 0.01 * t)
    h = jnp.dot(t, w2_ref[...], preferred_element_type=jnp.float32)
    hn_ref[...] = h * nsrc_ref[...]


def _final_body(agg_ref, ndst_ref, b2_ref, out_ref):
    a = agg_ref[0] + agg_ref[1]
    out_ref[...] = a * ndst_ref[...] + b2_ref[...]


_row_spec = pl.BlockSpec((R, D), lambda i: (i, 0))
_col_spec = pl.BlockSpec((R, 1), lambda i: (i, 0))
_deg_spec = pl.BlockSpec((NC, R, 1), lambda i: (0, i, 0))
_agg_spec = pl.BlockSpec((NC, R, D), lambda i: (0, i, 0))
_w_spec = pl.BlockSpec((D, D), lambda i: (0, 0))
_b_spec = pl.BlockSpec((1, D), lambda i: (0, 0))

_mm_call = pl.pallas_call(
    _mm_body,
    grid=(_GRID,),
    in_specs=[_row_spec, _w_spec],
    out_specs=_row_spec,
    out_shape=jax.ShapeDtypeStruct((N, D), jnp.float32),
)

_scale_call = pl.pallas_call(
    _scale_body,
    grid=(_GRID,),
    in_specs=[_row_spec, _deg_spec, _deg_spec],
    out_specs=[_row_spec, _col_spec, _col_spec],
    out_shape=[
        jax.ShapeDtypeStruct((N, D), jnp.float32),
        jax.ShapeDtypeStruct((NP, 1), jnp.float32),
        jax.ShapeDtypeStruct((NP, 1), jnp.float32),
    ],
)

_mid_call = pl.pallas_call(
    _mid_body,
    grid=(_GRID,),
    in_specs=[_agg_spec, _col_spec, _b_spec, _w_spec, _col_spec],
    out_specs=_row_spec,
    out_shape=jax.ShapeDtypeStruct((N, D), jnp.float32),
)

_final_call = pl.pallas_call(
    _final_body,
    grid=(_GRID,),
    in_specs=[_agg_spec, _col_spec, _b_spec],
    out_specs=_row_spec,
    out_shape=jax.ShapeDtypeStruct((N, D), jnp.float32),
)


def kernel(x, edge_index, W1, b1, W2, b2):
    deg_kernel, agg_kernel = _sc_kernels()
    src_r = edge_index[0].reshape(NW, NCHUNK, C)
    dst_r = edge_index[1].reshape(NW, NCHUNK, C)
    eidx = edge_index.reshape(2, NW, NCHUNK, C).transpose(1, 2, 0, 3)
    b1r = b1.reshape(1, D)
    b2r = b2.reshape(1, D)

    dego, degi = deg_kernel(src_r, dst_r)
    dego = dego.reshape(NC, NP, 1)
    degi = degi.reshape(NC, NP, 1)

    h1raw = _mm_call(x, W1)
    hn1, nsrc, ndst = _scale_call(h1raw, dego, degi)
    agg1 = agg_kernel(hn1, eidx)
    hn2 = _mid_call(agg1, ndst, b1r, W2, nsrc)
    agg2 = agg_kernel(hn2, eidx)
    return _final_call(agg2, ndst, b2r)


# speedup vs baseline: 1.0350x; 1.0350x over previous
import functools

import jax
import jax.numpy as jnp
from jax import lax
from jax.experimental import pallas as pl
from jax.experimental.pallas import tpu as pltpu
from jax.experimental.pallas import tpu_sc as plsc

N = 10000
E = 320000
D = 128

NC = 2
NS = 16
NW = NC * NS
EPW = E // NW
C = 80
NCHUNK = EPW // C
NP = 10240
DPT = NP // NS
RPT = NP // NS
RZ = 16


def _build_deg_kernel(mesh):
    @functools.partial(
        pl.kernel,
        out_type=[
            jax.ShapeDtypeStruct((NC, NP), jnp.float32),
            jax.ShapeDtypeStruct((NC, NP), jnp.float32),
        ],
        mesh=mesh,
        scratch_types=[
            pltpu.VMEM((NCHUNK, 2, C), jnp.int32),
            pltpu.VMEM((C,), jnp.float32),
            pltpu.VMEM((DPT,), jnp.float32),
            pltpu.VMEM_SHARED((NP,), jnp.float32),
            pltpu.VMEM_SHARED((NP,), jnp.float32),
            [pltpu.SemaphoreType.DMA for _ in range(3)],
        ],
    )
    def deg_kernel(eidx_hbm, dego_hbm, degi_hbm,
                   idx_v, ones_v, z_v, dego_sp, degi_sp, sd):
        c = lax.axis_index("c")
        s = lax.axis_index("s")
        wid = c * NS + s

        pltpu.async_copy(eidx_hbm.at[wid], idx_v, sd[2])

        for j in range(C // 16):
            ones_v[pl.ds(j * 16, 16)] = jnp.ones((16,), jnp.float32)

        def zero_body(i, _):
            z_v[pl.ds(i * 16, 16)] = jnp.zeros((16,), jnp.float32)
            return 0

        lax.fori_loop(0, DPT // 16, zero_body, 0)
        pltpu.sync_copy(z_v, dego_sp.at[pl.ds(s * DPT, DPT)])
        pltpu.sync_copy(z_v, degi_sp.at[pl.ds(s * DPT, DPT)])
        pltpu.make_async_copy(eidx_hbm.at[0], idx_v, sd[2]).wait()
        plsc.subcore_barrier()

        def scat2_start(i, p):
            pltpu.async_copy(ones_v, dego_sp.at[idx_v.at[i, 0]], sd[p],
                             add=True)
            pltpu.async_copy(ones_v, degi_sp.at[idx_v.at[i, 1]], sd[p],
                             add=True)

        def scat2_wait(p):
            pltpu.make_async_copy(ones_v, dego_sp.at[pl.ds(0, C)],
                                  sd[p]).wait()
            pltpu.make_async_copy(ones_v, degi_sp.at[pl.ds(0, C)],
                                  sd[p]).wait()

        scat2_start(0, 0)
        scat2_start(1, 1)

        def body(k, _):
            g = 2 * k
            scat2_wait(0)
            scat2_start(g + 2, 0)
            scat2_wait(1)
            scat2_start(g + 3, 1)
            return 0

        lax.fori_loop(0, (NCHUNK - 3) // 2, body, 0)
        scat2_wait(0)
        scat2_start(NCHUNK - 1, 0)
        scat2_wait(1)
        scat2_wait(0)
        plsc.subcore_barrier()

        pltpu.sync_copy(dego_sp.at[pl.ds(s * DPT, DPT)],
                        dego_hbm.at[c, pl.ds(s * DPT, DPT)])
        pltpu.sync_copy(degi_sp.at[pl.ds(s * DPT, DPT)],
                        degi_hbm.at[c, pl.ds(s * DPT, DPT)])

    return deg_kernel


def _build_agg_kernel(mesh):
    @functools.partial(
        pl.kernel,
        out_type=jax.ShapeDtypeStruct((NC, NP, D), jnp.float32),
        mesh=mesh,
        scratch_types=[
            [pltpu.VMEM((2, C), jnp.int32) for _ in range(4)],
            [pltpu.VMEM((C, D), jnp.float32) for _ in range(2)],
            pltpu.VMEM((RZ, D), jnp.float32),
            pltpu.VMEM_SHARED((NP, D), jnp.float32),
            [pltpu.SemaphoreType.DMA for _ in range(4)],
            [pltpu.SemaphoreType.DMA for _ in range(2)],
            [pltpu.SemaphoreType.DMA for _ in range(2)],
        ],
    )
    def agg_kernel(hn_hbm, eidx_hbm, out_hbm,
                   idx, rows, z_v, agg_sp, si, sg, ss):
        c = lax.axis_index("c")
        s = lax.axis_index("s")
        wid = c * NS + s

        def idx_start(i, r):
            pltpu.async_copy(eidx_hbm.at[wid, i], idx[r], si[r])

        def idx_wait(r):
            pltpu.make_async_copy(eidx_hbm.at[0, 0], idx[r], si[r]).wait()

        def gather_start(r_idx, p):
            pltpu.async_copy(hn_hbm.at[idx[r_idx].at[0]], rows[p], sg[p])

        def gather_wait(p):
            pltpu.make_async_copy(hn_hbm.at[pl.ds(0, C)], rows[p], sg[p]).wait()

        def scat_start(r_idx, p):
            pltpu.async_copy(rows[p], agg_sp.at[idx[r_idx].at[1]], ss[p],
                             add=True)

        def scat_wait(p):
            pltpu.make_async_copy(rows[p], agg_sp.at[pl.ds(0, C)],
                                  ss[p]).wait()

        def step(g, r, extra=0):
            p = r % 2
            scat_wait(1 - p)
            if extra:
                idx_start(g + 3, (r + 3) % 4)
            idx_wait((r + 1) % 4)
            gather_start((r + 1) % 4, 1 - p)
            gather_wait(p)
            scat_start(r, p)

        def zero_body(i, _):
            for j in range(D // 16):
                z_v[i, pl.ds(j * 16, 16)] = jnp.zeros((16,), jnp.float32)
            return 0

        lax.fori_loop(0, RZ, zero_body, 0)
        for k in range(RPT // RZ):
            pltpu.sync_copy(z_v, agg_sp.at[pl.ds(s * RPT + k * RZ, RZ)])
        plsc.subcore_barrier()

        idx_start(0, 0)
        idx_start(1, 1)
        idx_start(2, 2)
        idx_wait(0)
        gather_start(0, 0)
        idx_start(3, 3)
        idx_wait(1)
        gather_start(1, 1)
        gather_wait(0)
        scat_start(0, 0)
        step(1, 1, extra=1)
        step(2, 2, extra=1)
        step(3, 3, extra=1)

        def body(k, _):
            q = 4 * k
            step(q + 0, 0, extra=1)
            step(q + 1, 1, extra=1)
            step(q + 2, 2, extra=1)
            step(q + 3, 3, extra=1)
            return 0

        lax.fori_loop(1, (NCHUNK - 5) // 4, body, 0)
        step(NCHUNK - 5, 0, extra=1)
        step(NCHUNK - 4, 1, extra=1)
        step(NCHUNK - 3, 2)
        step(NCHUNK - 2, 3)
        scat_wait(1)
        gather_wait(0)
        scat_start(0, 0)
        scat_wait(0)

        plsc.subcore_barrier()
        pltpu.sync_copy(agg_sp.at[pl.ds(s * RPT, RPT)],
                        out_hbm.at[c, pl.ds(s * RPT, RPT)])

    return agg_kernel


@functools.cache
def _sc_kernels():
    mesh = plsc.VectorSubcoreMesh(
        core_axis_name="c", subcore_axis_name="s", num_cores=NC, num_subcores=NS
    )
    return _build_deg_kernel(mesh), _build_agg_kernel(mesh)


R = 2048
_GRID = NP // R


def _mm_body(x_ref, w1_ref, h_ref):
    h_ref[...] = jnp.dot(x_ref[...], w1_ref[...],
                         preferred_element_type=jnp.float32)


def _scale_body(h_ref, dego_ref, degi_ref, hn_ref, nsrc_ref, ndst_ref):
    dego = dego_ref[0] + dego_ref[1]
    degi = degi_ref[0] + degi_ref[1]
    nsrc = lax.rsqrt(jnp.maximum(dego, 1.0))
    ndst = lax.rsqrt(jnp.maximum(degi, 1.0))
    nsrc_ref[...] = nsrc
    ndst_ref[...] = ndst
    hn_ref[...] = h_ref[...] * nsrc


def _mid_body(agg_ref, ndst_ref, b1_ref, w2_ref, nsrc_ref, hn_ref):
    a = agg_ref[0] + agg_ref[1]
    t = a * ndst_ref[...] + b1_ref[...]
    t = jnp.where(t >= 0.0, t, 0.01 * t)
    h = jnp.dot(t, w2_ref[...], preferred_element_type=jnp.float32)
    hn_ref[...] = h * nsrc_ref[...]


def _final_body(agg_ref, ndst_ref, b2_ref, out_ref):
    a = agg_ref[0] + agg_ref[1]
    out_ref[...] = a * ndst_ref[...] + b2_ref[...]


_row_spec = pl.BlockSpec((R, D), lambda i: (i, 0))
_col_spec = pl.BlockSpec((R, 1), lambda i: (i, 0))
_deg_spec = pl.BlockSpec((NC, R, 1), lambda i: (0, i, 0))
_agg_spec = pl.BlockSpec((NC, R, D), lambda i: (0, i, 0))
_w_spec = pl.BlockSpec((D, D), lambda i: (0, 0))
_b_spec = pl.BlockSpec((1, D), lambda i: (0, 0))

_mm_call = pl.pallas_call(
    _mm_body,
    grid=(_GRID,),
    in_specs=[_row_spec, _w_spec],
    out_specs=_row_spec,
    out_shape=jax.ShapeDtypeStruct((N, D), jnp.float32),
)

_scale_call = pl.pallas_call(
    _scale_body,
    grid=(_GRID,),
    in_specs=[_row_spec, _deg_spec, _deg_spec],
    out_specs=[_row_spec, _col_spec, _col_spec],
    out_shape=[
        jax.ShapeDtypeStruct((N, D), jnp.float32),
        jax.ShapeDtypeStruct((NP, 1), jnp.float32),
        jax.ShapeDtypeStruct((NP, 1), jnp.float32),
    ],
)

_mid_call = pl.pallas_call(
    _mid_body,
    grid=(_GRID,),
    in_specs=[_agg_spec, _col_spec, _b_spec, _w_spec, _col_spec],
    out_specs=_row_spec,
    out_shape=jax.ShapeDtypeStruct((N, D), jnp.float32),
)

_final_call = pl.pallas_call(
    _final_body,
    grid=(_GRID,),
    in_specs=[_agg_spec, _col_spec, _b_spec],
    out_specs=_row_spec,
    out_shape=jax.ShapeDtypeStruct((N, D), jnp.float32),
)


def kernel(x, edge_index, W1, b1, W2, b2):
    deg_kernel, agg_kernel = _sc_kernels()
    eidx = edge_index.reshape(2, NW, NCHUNK, C).transpose(1, 2, 0, 3)
    b1r = b1.reshape(1, D)
    b2r = b2.reshape(1, D)

    dego, degi = deg_kernel(eidx)
    dego = dego.reshape(NC, NP, 1)
    degi = degi.reshape(NC, NP, 1)

    h1raw = _mm_call(x, W1)
    hn1, nsrc, ndst = _scale_call(h1raw, dego, degi)
    agg1 = agg_kernel(hn1, eidx)
    hn2 = _mid_call(agg1, ndst, b1r, W2, nsrc)
    agg2 = agg_kernel(hn2, eidx)
    return _final_call(agg2, ndst, b2r)


# speedup vs baseline: 1.1658x; 1.1263x over previous
import functools

import jax
import jax.numpy as jnp
from jax import lax
from jax.experimental import pallas as pl
from jax.experimental.pallas import tpu as pltpu
from jax.experimental.pallas import tpu_sc as plsc

N = 10000
E = 320000
D = 128

NC = 2
NS = 16
NW = NC * NS
EPW = E // NW
C = 80
NCHUNK = EPW // C
NP = 10240
DPT = NP // NS
RPT = NP // NS
RZ = 16


def _build_deg_kernel(mesh):
    @functools.partial(
        pl.kernel,
        out_type=[
            jax.ShapeDtypeStruct((NC, NP), jnp.float32),
            jax.ShapeDtypeStruct((NC, NP), jnp.float32),
        ],
        mesh=mesh,
        scratch_types=[
            pltpu.VMEM((NCHUNK, 2, C), jnp.int32),
            pltpu.VMEM((C,), jnp.float32),
            pltpu.VMEM((DPT,), jnp.float32),
            pltpu.VMEM_SHARED((NP,), jnp.float32),
            pltpu.VMEM_SHARED((NP,), jnp.float32),
            [pltpu.SemaphoreType.DMA for _ in range(3)],
        ],
    )
    def deg_kernel(eidx_hbm, dego_hbm, degi_hbm,
                   idx_v, ones_v, z_v, dego_sp, degi_sp, sd):
        c = lax.axis_index("c")
        s = lax.axis_index("s")
        wid = c * NS + s

        pltpu.async_copy(eidx_hbm.at[wid], idx_v, sd[2])

        for j in range(C // 16):
            ones_v[pl.ds(j * 16, 16)] = jnp.ones((16,), jnp.float32)

        def zero_body(i, _):
            z_v[pl.ds(i * 16, 16)] = jnp.zeros((16,), jnp.float32)
            return 0

        lax.fori_loop(0, DPT // 16, zero_body, 0)
        pltpu.sync_copy(z_v, dego_sp.at[pl.ds(s * DPT, DPT)])
        pltpu.sync_copy(z_v, degi_sp.at[pl.ds(s * DPT, DPT)])
        pltpu.make_async_copy(eidx_hbm.at[0], idx_v, sd[2]).wait()
        plsc.subcore_barrier()

        def scat2_start(i, p):
            pltpu.async_copy(ones_v, dego_sp.at[idx_v.at[i, 0]], sd[p],
                             add=True)
            pltpu.async_copy(ones_v, degi_sp.at[idx_v.at[i, 1]], sd[p],
                             add=True)

        def scat2_wait(p):
            pltpu.make_async_copy(ones_v, dego_sp.at[pl.ds(0, C)],
                                  sd[p]).wait()
            pltpu.make_async_copy(ones_v, degi_sp.at[pl.ds(0, C)],
                                  sd[p]).wait()

        scat2_start(0, 0)
        scat2_start(1, 1)

        def body(k, _):
            g = 2 * k
            scat2_wait(0)
            scat2_start(g + 2, 0)
            scat2_wait(1)
            scat2_start(g + 3, 1)
            return 0

        lax.fori_loop(0, (NCHUNK - 3) // 2, body, 0)
        scat2_wait(0)
        scat2_start(NCHUNK - 1, 0)
        scat2_wait(1)
        scat2_wait(0)
        plsc.subcore_barrier()

        pltpu.sync_copy(dego_sp.at[pl.ds(s * DPT, DPT)],
                        dego_hbm.at[c, pl.ds(s * DPT, DPT)])
        pltpu.sync_copy(degi_sp.at[pl.ds(s * DPT, DPT)],
                        degi_hbm.at[c, pl.ds(s * DPT, DPT)])

    return deg_kernel


def _build_agg_kernel(mesh):
    @functools.partial(
        pl.kernel,
        out_type=jax.ShapeDtypeStruct((NC, NP, D), jnp.float32),
        mesh=mesh,
        scratch_types=[
            [pltpu.VMEM((2, C), jnp.int32) for _ in range(6)],
            [pltpu.VMEM((C, D), jnp.float32) for _ in range(3)],
            pltpu.VMEM((RZ, D), jnp.float32),
            pltpu.VMEM_SHARED((NP, D), jnp.float32),
            [pltpu.SemaphoreType.DMA for _ in range(6)],
            [pltpu.SemaphoreType.DMA for _ in range(3)],
            [pltpu.SemaphoreType.DMA for _ in range(3)],
        ],
    )
    def agg_kernel(hn_hbm, eidx_hbm, out_hbm,
                   idx, rows, z_v, agg_sp, si, sg, ss):
        c = lax.axis_index("c")
        s = lax.axis_index("s")
        wid = c * NS + s

        def idx_start(i, r):
            pltpu.async_copy(eidx_hbm.at[wid, i], idx[r], si[r])

        def idx_wait(r):
            pltpu.make_async_copy(eidx_hbm.at[0, 0], idx[r], si[r]).wait()

        def gather_start(r_idx, p):
            pltpu.async_copy(hn_hbm.at[idx[r_idx].at[0]], rows[p], sg[p])

        def gather_wait(p):
            pltpu.make_async_copy(hn_hbm.at[pl.ds(0, C)], rows[p], sg[p]).wait()

        def scat_start(r_idx, p):
            pltpu.async_copy(rows[p], agg_sp.at[idx[r_idx].at[1]], ss[p],
                             add=True)

        def scat_wait(p):
            pltpu.make_async_copy(rows[p], agg_sp.at[pl.ds(0, C)],
                                  ss[p]).wait()

        def step(g, r, extra=True, last=False):
            p = r % 3
            pn = (r + 1) % 3
            scat_wait(pn)
            if extra:
                idx_start(g + 4, (r + 4) % 6)
            if not last:
                idx_wait((r + 1) % 6)
                gather_start((r + 1) % 6, pn)
            gather_wait(p)
            scat_start(r, p)

        def zero_body(i, _):
            for j in range(D // 16):
                z_v[i, pl.ds(j * 16, 16)] = jnp.zeros((16,), jnp.float32)
            return 0

        lax.fori_loop(0, RZ, zero_body, 0)
        for k in range(RPT // RZ):
            pltpu.sync_copy(z_v, agg_sp.at[pl.ds(s * RPT + k * RZ, RZ)])
        plsc.subcore_barrier()

        for r in range(5):
            idx_start(r, r)
        idx_wait(0)
        gather_start(0, 0)
        idx_wait(1)
        gather_start(1, 1)
        gather_wait(0)
        scat_start(0, 0)
        idx_start(5, 5)
        idx_wait(2)
        gather_start(2, 2)
        gather_wait(1)
        scat_start(1, 1)
        step(2, 2)
        step(3, 3)
        step(4, 4)
        step(5, 5)

        def body(k, _):
            q = 6 * k
            for r in range(6):
                step(q + r, r)
            return 0

        lax.fori_loop(1, (NCHUNK - 5) // 6, body, 0)
        step(NCHUNK - 5, 0)
        step(NCHUNK - 4, 1, extra=False)
        step(NCHUNK - 3, 2, extra=False)
        step(NCHUNK - 2, 3, extra=False)
        step(NCHUNK - 1, 4, extra=False, last=True)
        scat_wait(0)
        scat_wait(1)

        plsc.subcore_barrier()
        pltpu.sync_copy(agg_sp.at[pl.ds(s * RPT, RPT)],
                        out_hbm.at[c, pl.ds(s * RPT, RPT)])

    return agg_kernel


@functools.cache
def _sc_kernels():
    mesh = plsc.VectorSubcoreMesh(
        core_axis_name="c", subcore_axis_name="s", num_cores=NC, num_subcores=NS
    )
    return _build_deg_kernel(mesh), _build_agg_kernel(mesh)


R = 2048
_GRID = NP // R


def _mm_body(x_ref, w1_ref, h_ref):
    h_ref[...] = jnp.dot(x_ref[...], w1_ref[...],
                         preferred_element_type=jnp.float32)


def _scale_body(h_ref, dego_ref, degi_ref, hn_ref, nsrc_ref, ndst_ref):
    dego = dego_ref[0] + dego_ref[1]
    degi = degi_ref[0] + degi_ref[1]
    nsrc = lax.rsqrt(jnp.maximum(dego, 1.0))
    ndst = lax.rsqrt(jnp.maximum(degi, 1.0))
    nsrc_ref[...] = nsrc
    ndst_ref[...] = ndst
    hn_ref[...] = h_ref[...] * nsrc


def _mid_body(agg_ref, ndst_ref, b1_ref, w2_ref, nsrc_ref, hn_ref):
    a = agg_ref[0] + agg_ref[1]
    t = a * ndst_ref[...] + b1_ref[...]
    t = jnp.where(t >= 0.0, t, 0.01 * t)
    h = jnp.dot(t, w2_ref[...], preferred_element_type=jnp.float32)
    hn_ref[...] = h * nsrc_ref[...]


def _final_body(agg_ref, ndst_ref, b2_ref, out_ref):
    a = agg_ref[0] + agg_ref[1]
    out_ref[...] = a * ndst_ref[...] + b2_ref[...]


_row_spec = pl.BlockSpec((R, D), lambda i: (i, 0))
_col_spec = pl.BlockSpec((R, 1), lambda i: (i, 0))
_deg_spec = pl.BlockSpec((NC, R, 1), lambda i: (0, i, 0))
_agg_spec = pl.BlockSpec((NC, R, D), lambda i: (0, i, 0))
_w_spec = pl.BlockSpec((D, D), lambda i: (0, 0))
_b_spec = pl.BlockSpec((1, D), lambda i: (0, 0))

_mm_call = pl.pallas_call(
    _mm_body,
    grid=(_GRID,),
    in_specs=[_row_spec, _w_spec],
    out_specs=_row_spec,
    out_shape=jax.ShapeDtypeStruct((N, D), jnp.float32),
)

_scale_call = pl.pallas_call(
    _scale_body,
    grid=(_GRID,),
    in_specs=[_row_spec, _deg_spec, _deg_spec],
    out_specs=[_row_spec, _col_spec, _col_spec],
    out_shape=[
        jax.ShapeDtypeStruct((N, D), jnp.float32),
        jax.ShapeDtypeStruct((NP, 1), jnp.float32),
        jax.ShapeDtypeStruct((NP, 1), jnp.float32),
    ],
)

_mid_call = pl.pallas_call(
    _mid_body,
    grid=(_GRID,),
    in_specs=[_agg_spec, _col_spec, _b_spec, _w_spec, _col_spec],
    out_specs=_row_spec,
    out_shape=jax.ShapeDtypeStruct((N, D), jnp.float32),
)

_final_call = pl.pallas_call(
    _final_body,
    grid=(_GRID,),
    in_specs=[_agg_spec, _col_spec, _b_spec],
    out_specs=_row_spec,
    out_shape=jax.ShapeDtypeStruct((N, D), jnp.float32),
)


def kernel(x, edge_index, W1, b1, W2, b2):
    deg_kernel, agg_kernel = _sc_kernels()
    eidx = edge_index.reshape(2, NW, NCHUNK, C).transpose(1, 2, 0, 3)
    b1r = b1.reshape(1, D)
    b2r = b2.reshape(1, D)

    dego, degi = deg_kernel(eidx)
    dego = dego.reshape(NC, NP, 1)
    degi = degi.reshape(NC, NP, 1)

    h1raw = _mm_call(x, W1)
    hn1, nsrc, ndst = _scale_call(h1raw, dego, degi)
    agg1 = agg_kernel(hn1, eidx)
    hn2 = _mid_call(agg1, ndst, b1r, W2, nsrc)
    agg2 = agg_kernel(hn2, eidx)
    return _final_call(agg2, ndst, b2r)
